# baseline (speedup 1.0000x reference)
import functools

import jax
import jax.numpy as jnp
from jax import lax
from jax.experimental import pallas as pl
from jax.experimental.pallas import tpu as pltpu
from jax.experimental.pallas import tpu_sc as plsc

_B = 16
_N = 256
_TOTAL = _B * _N
_F = 15
_NC, _NS = 2, 16
_NW = _NC * _NS
_OPW = _TOTAL // _NW
_G = _OPW // 16

_REL = _B * _N * _N * 4
_REL_PW = _REL // _NW
_CH = 16384
_NCH = _REL_PW // _CH
_NB = 6

_mesh = plsc.VectorSubcoreMesh(
    core_axis_name="c", subcore_axis_name="s",
    num_cores=_NC, num_subcores=_NS)


@functools.partial(
    pl.kernel,
    out_type=(
        jax.ShapeDtypeStruct((_F, 32, 128), jnp.float32),
        jax.ShapeDtypeStruct((_REL,), jnp.float32),
    ),
    mesh=_mesh,
    compiler_params=pltpu.CompilerParams(needs_layout_passes=False),
    scratch_types=[
        pltpu.VMEM((_OPW,), jnp.int32),
        pltpu.VMEM((_OPW,), jnp.int32),
        pltpu.VMEM((_OPW,), jnp.int32),
        pltpu.VMEM((_OPW,), jnp.int32),
        pltpu.VMEM((_B,), jnp.int32),
        pltpu.VMEM((_F, _OPW), jnp.float32),
        [pltpu.VMEM((_CH,), jnp.float32) for _ in range(_NB)],
        pltpu.SemaphoreType.DMA,
        pltpu.SemaphoreType.DMA,
        pltpu.SemaphoreType.DMA,
        pltpu.SemaphoreType.DMA,
    ],
)
def _scene_gt_sc(col_hbm, mat_hbm, shp_hbm, siz_hbm, len_hbm, rel_hbm,
                 out_hbm, relo_hbm,
                 c_v, m_v, s_v, z_v, len_v, out_v, rbufs,
                 isem, osem, gsem, ssem):
    wid = lax.axis_index("s") * _NC + lax.axis_index("c")

    rbase = wid * _REL_PW
    gh = {}
    sh = {}
    for k in range(_NB):
        gh[k] = pltpu.async_copy(
            rel_hbm.at[pl.ds(rbase + k * _CH, _CH)], rbufs[k], gsem)

    base = wid * _OPW
    cps = [
        pltpu.async_copy(col_hbm.at[pl.ds(base, _OPW)], c_v, isem),
        pltpu.async_copy(mat_hbm.at[pl.ds(base, _OPW)], m_v, isem),
        pltpu.async_copy(shp_hbm.at[pl.ds(base, _OPW)], s_v, isem),
        pltpu.async_copy(siz_hbm.at[pl.ds(base, _OPW)], z_v, isem),
        pltpu.async_copy(len_hbm, len_v, isem),
    ]

    iota = lax.iota(jnp.int32, 16)
    zeros16 = jnp.zeros((16,), jnp.float32)
    ones16 = jnp.ones((16,), jnp.float32)
    for k in range(_F):
        for c8 in range(_OPW // 16):
            out_v[k, pl.ds(c8 * 16, 16)] = zeros16

    for cp in cps:
        cp.wait()

    scene = wid // 2
    len_scene = plsc.load_gather(len_v, [jnp.full((16,), scene, jnp.int32)])
    halfpos = (wid % 2) * _OPW

    for g in range(_G):
        sl = pl.ds(g * 16, 16)
        pos = halfpos + g * 16 + iota
        ones = jnp.where(pos < len_scene, ones16, zeros16)
        nloc = g * 16 + iota
        plsc.store_scatter(out_v, [c_v[sl] - 10, nloc], ones)
        plsc.store_scatter(out_v, [m_v[sl] - 12, nloc], ones)
        plsc.store_scatter(out_v, [s_v[sl] - 20, nloc], ones)
        plsc.store_scatter(out_v, [z_v[sl] - 27, nloc], ones)

    b = wid // 2
    r = (b // 8) * 16 + (wid % 2) * 8 + (b % 8)
    ocp = pltpu.async_copy(out_v, out_hbm.at[:, r, :], osem)

    for k in range(_NCH):
        gh[k].wait()
        sh[k] = pltpu.async_copy(
            rbufs[k % _NB], relo_hbm.at[pl.ds(rbase + k * _CH, _CH)], ssem)
        nk = k + _NB
        if nk < _NCH:
            sh[k].wait()
            gh[nk] = pltpu.async_copy(
                rel_hbm.at[pl.ds(rbase + nk * _CH, _CH)], rbufs[k % _NB], gsem)
    for k in range(max(0, _NCH - _NB), _NCH):
        sh[k].wait()
    ocp.wait()


def kernel(input, objects, objects_length, objects_color, objects_material,
           objects_shape, objects_size, relations_spatial_relation):
    rel_flat = (relations_spatial_relation
                .reshape(_B * _N * _N // 128, 128, 4)
                .transpose(0, 2, 1)
                .reshape(_REL))
    obj3, relo = _scene_gt_sc(objects_color, objects_material, objects_shape,
                              objects_size, objects_length, rel_flat)
    obj = (obj3.reshape(_F, 2, 2, 8, 128)
           .transpose(1, 3, 2, 4, 0)
           .reshape(_B, _N, _F))
    rel = (relo.reshape(_B * _N * _N // 128, 4, 128)
           .transpose(0, 2, 1)
           .reshape(_B, _N, _N, 4))
    return (obj, rel)


# speedup vs baseline: 1.0118x; 1.0118x over previous
import functools

import jax
import jax.numpy as jnp
from jax import lax
from jax.experimental import pallas as pl
from jax.experimental.pallas import tpu as pltpu
from jax.experimental.pallas import tpu_sc as plsc

_B = 16
_N = 256
_TOTAL = _B * _N
_F = 15
_NC, _NS = 2, 16
_NW = _NC * _NS
_OPW = _TOTAL // _NW
_G = _OPW // 16

_REL = _B * _N * _N * 4
_REL_PW = _REL // _NW
_CH = 32768
_NCH = _REL_PW // _CH
_NB = 3

_mesh = plsc.VectorSubcoreMesh(
    core_axis_name="c", subcore_axis_name="s",
    num_cores=_NC, num_subcores=_NS)


@functools.partial(
    pl.kernel,
    out_type=(
        jax.ShapeDtypeStruct((_F, 32, 128), jnp.float32),
        jax.ShapeDtypeStruct((_REL,), jnp.float32),
    ),
    mesh=_mesh,
    compiler_params=pltpu.CompilerParams(needs_layout_passes=False),
    scratch_types=[
        pltpu.VMEM((_OPW,), jnp.int32),
        pltpu.VMEM((_OPW,), jnp.int32),
        pltpu.VMEM((_OPW,), jnp.int32),
        pltpu.VMEM((_OPW,), jnp.int32),
        pltpu.VMEM((_B,), jnp.int32),
        pltpu.VMEM((_F, _OPW), jnp.float32),
        [pltpu.VMEM((_CH,), jnp.float32) for _ in range(_NB)],
        pltpu.SemaphoreType.DMA,
        pltpu.SemaphoreType.DMA,
        pltpu.SemaphoreType.DMA,
        pltpu.SemaphoreType.DMA,
    ],
)
def _scene_gt_sc(col_hbm, mat_hbm, shp_hbm, siz_hbm, len_hbm, rel_hbm,
                 out_hbm, relo_hbm,
                 c_v, m_v, s_v, z_v, len_v, out_v, rbufs,
                 isem, osem, gsem, ssem):
    wid = lax.axis_index("s") * _NC + lax.axis_index("c")

    rbase = wid * _REL_PW
    gh = {}
    sh = {}
    for k in range(_NB):
        gh[k] = pltpu.async_copy(
            rel_hbm.at[pl.ds(rbase + k * _CH, _CH)], rbufs[k], gsem)

    base = wid * _OPW
    cps = [
        pltpu.async_copy(col_hbm.at[pl.ds(base, _OPW)], c_v, isem),
        pltpu.async_copy(mat_hbm.at[pl.ds(base, _OPW)], m_v, isem),
        pltpu.async_copy(shp_hbm.at[pl.ds(base, _OPW)], s_v, isem),
        pltpu.async_copy(siz_hbm.at[pl.ds(base, _OPW)], z_v, isem),
        pltpu.async_copy(len_hbm, len_v, isem),
    ]

    iota = lax.iota(jnp.int32, 16)
    zeros16 = jnp.zeros((16,), jnp.float32)
    ones16 = jnp.ones((16,), jnp.float32)
    for k in range(_F):
        for c8 in range(_OPW // 16):
            out_v[k, pl.ds(c8 * 16, 16)] = zeros16

    for cp in cps:
        cp.wait()

    scene = wid // 2
    len_scene = plsc.load_gather(len_v, [jnp.full((16,), scene, jnp.int32)])
    halfpos = (wid % 2) * _OPW

    for g in range(_G):
        sl = pl.ds(g * 16, 16)
        pos = halfpos + g * 16 + iota
        ones = jnp.where(pos < len_scene, ones16, zeros16)
        nloc = g * 16 + iota
        plsc.store_scatter(out_v, [c_v[sl] - 10, nloc], ones)
        plsc.store_scatter(out_v, [m_v[sl] - 12, nloc], ones)
        plsc.store_scatter(out_v, [s_v[sl] - 20, nloc], ones)
        plsc.store_scatter(out_v, [z_v[sl] - 27, nloc], ones)

    b = wid // 2
    r = (b // 8) * 16 + (wid % 2) * 8 + (b % 8)
    ocp = pltpu.async_copy(out_v, out_hbm.at[:, r, :], osem)

    for k in range(_NCH):
        gh[k].wait()
        sh[k] = pltpu.async_copy(
            rbufs[k % _NB], relo_hbm.at[pl.ds(rbase + k * _CH, _CH)], ssem)
        nk = k + _NB
        if nk < _NCH:
            sh[k].wait()
            gh[nk] = pltpu.async_copy(
                rel_hbm.at[pl.ds(rbase + nk * _CH, _CH)], rbufs[k % _NB], gsem)
    for k in range(max(0, _NCH - _NB), _NCH):
        sh[k].wait()
    ocp.wait()


def kernel(input, objects, objects_length, objects_color, objects_material,
           objects_shape, objects_size, relations_spatial_relation):
    rel_flat = (relations_spatial_relation
                .reshape(_B * _N * _N // 128, 128, 4)
                .transpose(0, 2, 1)
                .reshape(_REL))
    obj3, relo = _scene_gt_sc(objects_color, objects_material, objects_shape,
                              objects_size, objects_length, rel_flat)
    obj = (obj3.reshape(_F, 2, 2, 8, 128)
           .transpose(1, 3, 2, 4, 0)
           .reshape(_B, _N, _F))
    rel = (relo.reshape(_B * _N * _N // 128, 4, 128)
           .transpose(0, 2, 1)
           .reshape(_B, _N, _N, 4))
    return (obj, rel)
